# baseline (speedup 1.0000x reference)
import functools

import jax
import jax.numpy as jnp
from jax import lax
from jax.experimental import pallas as pl
from jax.experimental.pallas import tpu as pltpu
from jax.experimental.pallas import tpu_sc as plsc

N_NODES = 10000
N_EDGES = 320000
D_FEAT = 128

NC = 2
NS = 16
L = 16
NW = NC * NS
EPW = N_EDGES // NW
CH = 40
NCHUNK = EPW // CH
G = 50
NGRP = NCHUNK // G
NBUF = 6
A = 4
DEG_CH = 80
DEG_NCH = EPW // DEG_CH
NPAD = 10240
RPT = NPAD // NS
DSUB = D_FEAT // L
DEG_W = 128

_MESH = plsc.VectorSubcoreMesh(
    core_axis_name="c", subcore_axis_name="s", num_cores=NC, num_subcores=NS
)


def _deg_partials(dst_r):
    @functools.partial(
        pl.kernel,
        out_type=jax.ShapeDtypeStruct((NC, NPAD, DEG_W), jnp.float32),
        mesh=_MESH,
        scratch_types=[
            pltpu.VMEM((DEG_NCH, DEG_CH), jnp.int32),
            pltpu.VMEM((DEG_CH, DEG_W), jnp.float32),
            pltpu.SemaphoreType.DMA,
            pltpu.VMEM_SHARED((NPAD, DEG_W), jnp.float32),
        ],
    )
    def k(dst_hbm, out_hbm, dall, ones_v, sem, acc_sh):
        cid = lax.axis_index("c")
        sid = lax.axis_index("s")
        wid = sid * NC + cid
        one = jnp.full((L,), 1.0, jnp.float32)
        zero = jnp.zeros((L,), jnp.float32)

        def fill(val):
            def fill_row(i, carry):
                for j in range(DEG_W // L):
                    ones_v[i, pl.ds(j * L, L)] = val
                return carry

            lax.fori_loop(0, DEG_CH, fill_row, 0)

        fill(zero)
        pltpu.sync_copy(dst_hbm.at[wid], dall)
        base = sid * RPT
        zd = [
            pltpu.async_copy(
                ones_v, acc_sh.at[pl.ds(base + t * DEG_CH, DEG_CH)], sem
            )
            for t in range(RPT // DEG_CH)
        ]
        for d in zd:
            d.wait()
        fill(one)
        plsc.subcore_barrier()

        DW = 48

        def issue(j):
            pltpu.async_copy(ones_v, acc_sh.at[dall.at[j]], sem, add=True)

        def drain_one():
            pltpu.make_async_copy(ones_v, acc_sh.at[dall.at[0]], sem).wait()

        def prologue(j, carry):
            issue(j)
            return carry

        lax.fori_loop(0, DW, prologue, 0)

        def steady(j, carry):
            issue(j)
            drain_one()
            return carry

        lax.fori_loop(DW, DEG_NCH, steady, 0)

        def epilogue(j, carry):
            drain_one()
            return carry

        lax.fori_loop(0, DW, epilogue, 0)
        plsc.subcore_barrier()
        pltpu.sync_copy(
            acc_sh.at[pl.ds(base, RPT)], out_hbm.at[cid].at[pl.ds(base, RPT)]
        )

    return k(dst_r)


def _aggregate(src_r, dst_r, y):
    @functools.partial(
        pl.kernel,
        out_type=jax.ShapeDtypeStruct((NC, NPAD, D_FEAT), jnp.float32),
        mesh=_MESH,
        scratch_types=[
            pltpu.VMEM((G, CH), jnp.int32),
            pltpu.VMEM((G, CH), jnp.int32),
            pltpu.VMEM((NBUF * CH, D_FEAT), jnp.float32),
            pltpu.SemaphoreType.DMA,
            pltpu.SemaphoreType.DMA,
            pltpu.VMEM_SHARED((NPAD, D_FEAT), jnp.float32),
        ],
    )
    def k(src_hbm, dst_hbm, y_hbm, out_hbm, sgrp, dgrp, rows, gsem, ssem, acc_sh):
        cid = lax.axis_index("c")
        sid = lax.axis_index("s")
        wid = sid * NC + cid
        zero = jnp.zeros((L,), jnp.float32)

        def fill_zeros(i, carry):
            for j in range(DSUB):
                rows[i, pl.ds(j * L, L)] = zero
            return carry

        lax.fori_loop(0, CH, fill_zeros, 0)

        base = sid * RPT
        zstage = rows.at[pl.ds(0, CH)]
        zd = [
            pltpu.async_copy(zstage, acc_sh.at[pl.ds(base + t * CH, CH)], gsem)
            for t in range(RPT // CH)
        ]
        for d in zd:
            d.wait()
        plsc.subcore_barrier()

        def half(j):
            return rows.at[pl.ds((j % NBUF) * CH, CH)]

        def group(g, carry):
            pltpu.sync_copy(src_hbm.at[wid, g], sgrp)
            pltpu.sync_copy(dst_hbm.at[wid, g], dgrp)

            def gather(j):
                return pltpu.async_copy(y_hbm.at[sgrp.at[j]], half(j), gsem)

            def scatter(j):
                return pltpu.async_copy(
                    half(j), acc_sh.at[dgrp.at[j]], ssem, add=True
                )

            gd = [None] * G
            sd = [None] * G
            for a in range(A):
                gd[a] = gather(a)
            for j in range(G):
                gd[j].wait()
                h = j + A - NBUF
                if h >= 0:
                    sd[h].wait()
                if j + A < G:
                    gd[j + A] = gather(j + A)
                sd[j] = scatter(j)
            for j in range(G - NBUF + A, G):
                sd[j].wait()
            return carry

        lax.fori_loop(0, NGRP, group, 0)
        plsc.subcore_barrier()
        pltpu.sync_copy(
            acc_sh.at[pl.ds(base, RPT)], out_hbm.at[cid].at[pl.ds(base, RPT)]
        )

    return k(src_r, dst_r, y)


_BR = 1000


def _tc_specs():
    deg_spec = pl.BlockSpec((NC, _BR, DEG_W), lambda i: (0, i, 0))
    row_spec = pl.BlockSpec((_BR, D_FEAT), lambda i: (i, 0))
    part_spec = pl.BlockSpec((NC, _BR, D_FEAT), lambda i: (0, i, 0))
    return deg_spec, row_spec, part_spec


def _prescale(degp, x):
    deg_spec, row_spec, _ = _tc_specs()

    def body(dp_ref, x_ref, y_ref):
        deg = dp_ref[0, :, 0:1] + dp_ref[1, :, 0:1]
        y_ref[:, :] = lax.rsqrt(1.0 + deg) * x_ref[:, :]

    return pl.pallas_call(
        body,
        grid=(N_NODES // _BR,),
        in_specs=[deg_spec, row_spec],
        out_specs=row_spec,
        out_shape=jax.ShapeDtypeStruct((N_NODES, D_FEAT), jnp.float32),
    )(degp, x)


def _mid_combine(degp, parts, y):
    deg_spec, row_spec, part_spec = _tc_specs()

    def body(dp_ref, p_ref, y_ref, o_ref):
        deg = dp_ref[0, :, 0:1] + dp_ref[1, :, 0:1]
        s = p_ref[0] + p_ref[1] + y_ref[:, :]
        o_ref[:, :] = s / (1.0 + deg)

    return pl.pallas_call(
        body,
        grid=(N_NODES // _BR,),
        in_specs=[deg_spec, part_spec, row_spec],
        out_specs=row_spec,
        out_shape=jax.ShapeDtypeStruct((N_NODES, D_FEAT), jnp.float32),
    )(degp, parts, y)


def _final_combine(degp, parts, y):
    deg_spec, row_spec, part_spec = _tc_specs()

    def body(dp_ref, p_ref, y_ref, o_ref):
        deg = dp_ref[0, :, 0:1] + dp_ref[1, :, 0:1]
        s = p_ref[0] + p_ref[1] + y_ref[:, :]
        o_ref[:, :] = lax.rsqrt(1.0 + deg) * s

    return pl.pallas_call(
        body,
        grid=(N_NODES // _BR,),
        in_specs=[deg_spec, part_spec, row_spec],
        out_specs=row_spec,
        out_shape=jax.ShapeDtypeStruct((N_NODES, D_FEAT), jnp.float32),
    )(degp, parts, y)


@jax.jit
def kernel(x, edge_index):
    dst = edge_index[0].astype(jnp.int32)
    src = edge_index[1].astype(jnp.int32)
    dst_r = dst.reshape(NW, NGRP, G, CH)
    src_r = src.reshape(NW, NGRP, G, CH)
    dst_d = dst.reshape(NW, DEG_NCH, DEG_CH)

    degp = _deg_partials(dst_d)
    y1 = _prescale(degp, x)
    p1 = _aggregate(src_r, dst_r, y1)
    y2 = _mid_combine(degp, p1, y1)
    p2 = _aggregate(src_r, dst_r, y2)
    out = _final_combine(degp, p2, y2)
    return out


# speedup vs baseline: 1.0032x; 1.0032x over previous
import functools

import jax
import jax.numpy as jnp
from jax import lax
from jax.experimental import pallas as pl
from jax.experimental.pallas import tpu as pltpu
from jax.experimental.pallas import tpu_sc as plsc

N_NODES = 10000
N_EDGES = 320000
D_FEAT = 128

NC = 2
NS = 16
L = 16
NW = NC * NS
EPW = N_EDGES // NW
CH = 40
NCHUNK = EPW // CH
G = 50
NGRP = NCHUNK // G
NBUF = 6
A = 4
DEG_CH = 80
DEG_NCH = EPW // DEG_CH
NPAD = 10240
RPT = NPAD // NS
DSUB = D_FEAT // L
DEG_W = 128

_MESH = plsc.VectorSubcoreMesh(
    core_axis_name="c", subcore_axis_name="s", num_cores=NC, num_subcores=NS
)


def _deg_partials(dst_r):
    @functools.partial(
        pl.kernel,
        out_type=jax.ShapeDtypeStruct((NC, NPAD, DEG_W), jnp.float32),
        mesh=_MESH,
        scratch_types=[
            pltpu.VMEM((DEG_NCH, DEG_CH), jnp.int32),
            pltpu.VMEM((DEG_CH, DEG_W), jnp.float32),
            pltpu.SemaphoreType.DMA,
            pltpu.VMEM_SHARED((NPAD, DEG_W), jnp.float32),
        ],
    )
    def k(dst_hbm, out_hbm, dall, ones_v, sem, acc_sh):
        cid = lax.axis_index("c")
        sid = lax.axis_index("s")
        wid = sid * NC + cid
        one = jnp.full((L,), 1.0, jnp.float32)
        zero = jnp.zeros((L,), jnp.float32)

        def fill(val):
            def fill_row(i, carry):
                for j in range(DEG_W // L):
                    ones_v[i, pl.ds(j * L, L)] = val
                return carry

            lax.fori_loop(0, DEG_CH, fill_row, 0)

        fill(zero)
        pltpu.sync_copy(dst_hbm.at[wid], dall)
        base = sid * RPT
        zd = [
            pltpu.async_copy(
                ones_v, acc_sh.at[pl.ds(base + t * DEG_CH, DEG_CH)], sem
            )
            for t in range(RPT // DEG_CH)
        ]
        for d in zd:
            d.wait()
        fill(one)
        plsc.subcore_barrier()

        DW = 24

        def issue(j):
            pltpu.async_copy(ones_v, acc_sh.at[dall.at[j]], sem, add=True)

        def drain_one():
            pltpu.make_async_copy(ones_v, acc_sh.at[dall.at[0]], sem).wait()

        def prologue(j, carry):
            issue(j)
            return carry

        lax.fori_loop(0, DW, prologue, 0)

        def steady(j, carry):
            issue(j)
            drain_one()
            return carry

        lax.fori_loop(DW, DEG_NCH, steady, 0)

        def epilogue(j, carry):
            drain_one()
            return carry

        lax.fori_loop(0, DW, epilogue, 0)
        plsc.subcore_barrier()
        pltpu.sync_copy(
            acc_sh.at[pl.ds(base, RPT)], out_hbm.at[cid].at[pl.ds(base, RPT)]
        )

    return k(dst_r)


def _aggregate(src_r, dst_r, y):
    @functools.partial(
        pl.kernel,
        out_type=jax.ShapeDtypeStruct((NC, NPAD, D_FEAT), jnp.float32),
        mesh=_MESH,
        scratch_types=[
            pltpu.VMEM((G, CH), jnp.int32),
            pltpu.VMEM((G, CH), jnp.int32),
            pltpu.VMEM((NBUF * CH, D_FEAT), jnp.float32),
            pltpu.SemaphoreType.DMA,
            pltpu.SemaphoreType.DMA,
            pltpu.VMEM_SHARED((NPAD, D_FEAT), jnp.float32),
        ],
    )
    def k(src_hbm, dst_hbm, y_hbm, out_hbm, sgrp, dgrp, rows, gsem, ssem, acc_sh):
        cid = lax.axis_index("c")
        sid = lax.axis_index("s")
        wid = sid * NC + cid
        zero = jnp.zeros((L,), jnp.float32)

        def fill_zeros(i, carry):
            for j in range(DSUB):
                rows[i, pl.ds(j * L, L)] = zero
            return carry

        lax.fori_loop(0, CH, fill_zeros, 0)

        base = sid * RPT
        zstage = rows.at[pl.ds(0, CH)]
        zd = [
            pltpu.async_copy(zstage, acc_sh.at[pl.ds(base + t * CH, CH)], gsem)
            for t in range(RPT // CH)
        ]
        for d in zd:
            d.wait()
        plsc.subcore_barrier()

        def half(j):
            return rows.at[pl.ds((j % NBUF) * CH, CH)]

        def group(g, carry):
            pltpu.sync_copy(src_hbm.at[wid, g], sgrp)
            pltpu.sync_copy(dst_hbm.at[wid, g], dgrp)

            def gather(j):
                return pltpu.async_copy(y_hbm.at[sgrp.at[j]], half(j), gsem)

            def scatter(j):
                return pltpu.async_copy(
                    half(j), acc_sh.at[dgrp.at[j]], ssem, add=True
                )

            gd = [None] * G
            sd = [None] * G
            for a in range(A):
                gd[a] = gather(a)
            for j in range(G):
                gd[j].wait()
                h = j + A - NBUF
                if h >= 0:
                    sd[h].wait()
                if j + A < G:
                    gd[j + A] = gather(j + A)
                sd[j] = scatter(j)
            for j in range(G - NBUF + A, G):
                sd[j].wait()
            return carry

        lax.fori_loop(0, NGRP, group, 0)
        plsc.subcore_barrier()
        pltpu.sync_copy(
            acc_sh.at[pl.ds(base, RPT)], out_hbm.at[cid].at[pl.ds(base, RPT)]
        )

    return k(src_r, dst_r, y)


_BR = 1000


def _tc_specs():
    deg_spec = pl.BlockSpec((NC, _BR, DEG_W), lambda i: (0, i, 0))
    row_spec = pl.BlockSpec((_BR, D_FEAT), lambda i: (i, 0))
    part_spec = pl.BlockSpec((NC, _BR, D_FEAT), lambda i: (0, i, 0))
    return deg_spec, row_spec, part_spec


def _prescale(degp, x):
    deg_spec, row_spec, _ = _tc_specs()

    def body(dp_ref, x_ref, y_ref):
        deg = dp_ref[0, :, 0:1] + dp_ref[1, :, 0:1]
        y_ref[:, :] = lax.rsqrt(1.0 + deg) * x_ref[:, :]

    return pl.pallas_call(
        body,
        grid=(N_NODES // _BR,),
        in_specs=[deg_spec, row_spec],
        out_specs=row_spec,
        out_shape=jax.ShapeDtypeStruct((N_NODES, D_FEAT), jnp.float32),
    )(degp, x)


def _mid_combine(degp, parts, y):
    deg_spec, row_spec, part_spec = _tc_specs()

    def body(dp_ref, p_ref, y_ref, o_ref):
        deg = dp_ref[0, :, 0:1] + dp_ref[1, :, 0:1]
        s = p_ref[0] + p_ref[1] + y_ref[:, :]
        o_ref[:, :] = s / (1.0 + deg)

    return pl.pallas_call(
        body,
        grid=(N_NODES // _BR,),
        in_specs=[deg_spec, part_spec, row_spec],
        out_specs=row_spec,
        out_shape=jax.ShapeDtypeStruct((N_NODES, D_FEAT), jnp.float32),
    )(degp, parts, y)


def _final_combine(degp, parts, y):
    deg_spec, row_spec, part_spec = _tc_specs()

    def body(dp_ref, p_ref, y_ref, o_ref):
        deg = dp_ref[0, :, 0:1] + dp_ref[1, :, 0:1]
        s = p_ref[0] + p_ref[1] + y_ref[:, :]
        o_ref[:, :] = lax.rsqrt(1.0 + deg) * s

    return pl.pallas_call(
        body,
        grid=(N_NODES // _BR,),
        in_specs=[deg_spec, part_spec, row_spec],
        out_specs=row_spec,
        out_shape=jax.ShapeDtypeStruct((N_NODES, D_FEAT), jnp.float32),
    )(degp, parts, y)


@jax.jit
def kernel(x, edge_index):
    dst = edge_index[0].astype(jnp.int32)
    src = edge_index[1].astype(jnp.int32)
    dst_r = dst.reshape(NW, NGRP, G, CH)
    src_r = src.reshape(NW, NGRP, G, CH)
    dst_d = dst.reshape(NW, DEG_NCH, DEG_CH)

    degp = _deg_partials(dst_d)
    y1 = _prescale(degp, x)
    p1 = _aggregate(src_r, dst_r, y1)
    y2 = _mid_combine(degp, p1, y1)
    p2 = _aggregate(src_r, dst_r, y2)
    out = _final_combine(degp, p2, y2)
    return out
